# Initial kernel scaffold; baseline (speedup 1.0000x reference)
import jax
import jax.numpy as jnp
from jax.experimental import pallas as pl


def kernel(x, edge_index, W_l, b_l, W_r, W_d1, b_d1, W_d2, b_d2):
    raise NotImplementedError("write your pallas kernel here")


# speedup vs baseline: 3.1295x; 3.1295x over previous
import functools
import jax
import jax.numpy as jnp
from jax import lax
from jax.experimental import pallas as pl
from jax.experimental.pallas import tpu as pltpu
from jax.experimental.pallas import tpu_sc as plsc

_N = 10000
_NP = 10240
_D = 256
_HALF = 128
_W = 144
_CH = 128
_TILES = 16
_ROWS_PER_TILE = _NP // _TILES


def _sc_segment_sum(xflat, srcp, dstp, zrows, nchunks):
    edges_per_tile = nchunks * _CH

    mesh = plsc.VectorSubcoreMesh(core_axis_name="c", subcore_axis_name="s")

    @functools.partial(
        pl.kernel,
        out_type=jax.ShapeDtypeStruct((2, _NP, _W), jnp.float32),
        mesh=mesh,
        compiler_params=pltpu.CompilerParams(use_tc_tiling_on_sc=False),
        scratch_types=[
            pltpu.VMEM((_CH,), jnp.int32),
            pltpu.VMEM((_CH,), jnp.int32),
            pltpu.VMEM((_CH, _W), jnp.float32),
            pltpu.SemaphoreType.DMA,
            pltpu.VMEM_SHARED((_NP, _W), jnp.float32),
        ],
    )
    def k(xflat_hbm, src_hbm, dst_hbm, z_hbm, out_hbm,
          src_v, dst_v, rows_v, sem, agg_sh):
        cid = lax.axis_index("c")
        sid = lax.axis_index("s")
        row0 = sid * _ROWS_PER_TILE
        pltpu.sync_copy(z_hbm, agg_sh.at[pl.ds(row0, _ROWS_PER_TILE)])
        plsc.subcore_barrier()

        base0 = sid * edges_per_tile
        off = cid * _NP

        def chunk(c, carry):
            b = base0 + c * _CH
            pltpu.sync_copy(src_hbm.at[pl.ds(b, _CH)], src_v)
            pltpu.sync_copy(dst_hbm.at[pl.ds(b, _CH)], dst_v)
            for i in range(_CH // 16):
                s = pl.ds(i * 16, 16)
                src_v[s] = src_v[s] + off
            pltpu.async_copy(xflat_hbm.at[src_v], rows_v, sem).wait()
            pltpu.sync_copy(rows_v, agg_sh.at[dst_v], add=True)
            return carry

        lax.fori_loop(0, nchunks, chunk, 0)
        plsc.subcore_barrier()
        pltpu.sync_copy(agg_sh.at[pl.ds(row0, _ROWS_PER_TILE)],
                        out_hbm.at[cid, pl.ds(row0, _ROWS_PER_TILE)])

    return k(xflat, srcp, dstp, zrows)


def _tc_body(a0, a1, x, wl, bl, wr, wd1, bd1, wd2, bd2, o):
    deg = jnp.clip(a0[:, _HALF:_HALF + 1], 1.0, None)
    agg = jnp.concatenate([a0[:, :_HALF], a1[:, :_HALF]], axis=1) / deg
    h = jnp.dot(agg, wl[...], preferred_element_type=jnp.float32)
    h += jnp.dot(x[...], wr[...], preferred_element_type=jnp.float32)
    h = jnp.maximum(h + bl[...], 0.0)
    hd = jnp.dot(h, wd1[...], preferred_element_type=jnp.float32)
    hd = jnp.maximum(hd + bd1[...], 0.0)
    out = jnp.dot(hd, wd2[...], preferred_element_type=jnp.float32)
    o[...] = out + bd2[...]


def _tc_dense(a0, a1, x, W_l, b_l, W_r, W_d1, b_d1, W_d2, b_d2):
    n = x.shape[0]
    B = 512
    grid = (pl.cdiv(_NP, B),)

    def row_blk(cols):
        return pl.BlockSpec((B, cols), lambda i: (i, 0))

    def full(shape):
        return pl.BlockSpec(shape, lambda i: tuple(0 for _ in shape))

    return pl.pallas_call(
        _tc_body,
        grid=grid,
        in_specs=[
            row_blk(_W), row_blk(_W), row_blk(_D),
            full(W_l.shape), full(b_l.shape), full(W_r.shape),
            full(W_d1.shape), full(b_d1.shape),
            full(W_d2.shape), full(b_d2.shape),
        ],
        out_specs=row_blk(_D),
        out_shape=jax.ShapeDtypeStruct((n, _D), jnp.float32),
    )(a0, a1, x, W_l, b_l, W_r, W_d1, b_d1, W_d2, b_d2)


def kernel(x, edge_index, W_l, b_l, W_r, W_d1, b_d1, W_d2, b_d2):
    n = x.shape[0]
    e = edge_index.shape[1]
    f32 = jnp.float32

    src = edge_index[0].astype(jnp.int32)
    dst = edge_index[1].astype(jnp.int32)

    ones = jnp.ones((n, 1), f32)
    padc = jnp.zeros((n, _W - _HALF - 1), f32)
    padr = jnp.zeros((_NP - n, _W), f32)
    xflat = jnp.concatenate([
        jnp.concatenate([x[:, :_HALF], ones, padc], axis=1), padr,
        jnp.concatenate([x[:, _HALF:], ones, padc], axis=1), padr,
    ], axis=0)

    step = _TILES * _CH
    ep = ((e + step - 1) // step) * step
    nchunks = ep // step
    srcp = jnp.concatenate([src, jnp.zeros((ep - e,), jnp.int32)])
    dstp = jnp.concatenate([dst, jnp.full((ep - e,), n, jnp.int32)])
    zrows = jnp.zeros((_ROWS_PER_TILE, _W), f32)

    agg2 = _sc_segment_sum(xflat, srcp, dstp, zrows, nchunks)

    b_l2 = b_l.reshape(1, -1)
    b_d12 = b_d1.reshape(1, -1)
    b_d22 = b_d2.reshape(1, -1)
    return _tc_dense(agg2[0], agg2[1], x, W_l, b_l2, W_r,
                     W_d1, b_d12, W_d2, b_d22)


# speedup vs baseline: 3.2545x; 1.0400x over previous
import functools
import jax
import jax.numpy as jnp
from jax import lax
from jax.experimental import pallas as pl
from jax.experimental.pallas import tpu as pltpu
from jax.experimental.pallas import tpu_sc as plsc

_N = 10000
_NP = 10016
_D = 256
_HALF = 128
_W = 144
_CH = 128
_K = 4
_TILES = 16
_ROWS_PER_TILE = _NP // _TILES


def _sc_segment_sum(xflat, srcs2, dsts, zrows, nchunks):
    mesh = plsc.VectorSubcoreMesh(core_axis_name="c", subcore_axis_name="s")
    nsup = nchunks // _K

    @functools.partial(
        pl.kernel,
        out_type=jax.ShapeDtypeStruct((2, _NP, _W), jnp.float32),
        mesh=mesh,
        compiler_params=pltpu.CompilerParams(use_tc_tiling_on_sc=False),
        scratch_types=[
            pltpu.VMEM((2, _K, _CH), jnp.int32),
            pltpu.VMEM((2, _K, _CH), jnp.int32),
            pltpu.VMEM((_CH, _W), jnp.float32),
            pltpu.VMEM((_CH, _W), jnp.float32),
            pltpu.SemaphoreType.DMA,
            pltpu.SemaphoreType.DMA,
            pltpu.VMEM_SHARED((_NP, _W), jnp.float32),
        ],
    )
    def k(xflat_hbm, src_hbm, dst_hbm, z_hbm, out_hbm,
          src_sup, dst_sup, rows0, rows1, sem0, sem1, agg_sh):
        cid = lax.axis_index("c")
        sid = lax.axis_index("s")
        row0 = sid * _ROWS_PER_TILE
        rows_v = (rows0, rows1)
        sems = (sem0, sem1)
        crow0 = sid * nchunks

        def load_sup(s, sb):
            pltpu.sync_copy(src_hbm.at[cid, pl.ds(crow0 + s * _K, _K)],
                            src_sup.at[sb])
            pltpu.sync_copy(dst_hbm.at[pl.ds(crow0 + s * _K, _K)],
                            dst_sup.at[sb])

        def fire(sb, j, b):
            pltpu.async_copy(xflat_hbm.at[src_sup.at[sb, j]],
                             rows_v[b], sems[b])

        def wait(sb, j, b):
            pltpu.make_async_copy(xflat_hbm.at[src_sup.at[sb, j]],
                                  rows_v[b], sems[b]).wait()

        pltpu.sync_copy(z_hbm, agg_sh.at[pl.ds(row0, _ROWS_PER_TILE)])
        plsc.subcore_barrier()

        load_sup(0, 0)
        fire(0, 0, 0)

        def body(i, carry):
            for sp in range(2):
                s = 2 * i + sp
                sb = sp
                load_sup(s + 1, 1 - sb)
                for j in range(_K):
                    b = j % 2
                    jn, sbn = (j + 1, sb) if j + 1 < _K else (0, 1 - sb)
                    fire(sbn, jn, 1 - b)
                    wait(sb, j, b)
                    pltpu.sync_copy(rows_v[b],
                                    agg_sh.at[dst_sup.at[sb, j]], add=True)
            return carry

        lax.fori_loop(0, nsup // 2, body, 0)
        wait(0, 0, 0)

        plsc.subcore_barrier()
        pltpu.sync_copy(agg_sh.at[pl.ds(row0, _ROWS_PER_TILE)],
                        out_hbm.at[cid, pl.ds(row0, _ROWS_PER_TILE)])

    return k(xflat, srcs2, dsts, zrows)


def _tc_body(a0, a1, x, wl, bl, wr, wd1, bd1, wd2, bd2, o):
    deg = jnp.clip(a0[:, _HALF:_HALF + 1], 1.0, None)
    agg = jnp.concatenate([a0[:, :_HALF], a1[:, :_HALF]], axis=1) / deg
    h = jnp.dot(agg, wl[...], preferred_element_type=jnp.float32)
    h += jnp.dot(x[...], wr[...], preferred_element_type=jnp.float32)
    h = jnp.maximum(h + bl[...], 0.0)
    hd = jnp.dot(h, wd1[...], preferred_element_type=jnp.float32)
    hd = jnp.maximum(hd + bd1[...], 0.0)
    out = jnp.dot(hd, wd2[...], preferred_element_type=jnp.float32)
    o[...] = out + bd2[...]


def _tc_dense(a0, a1, x, W_l, b_l, W_r, W_d1, b_d1, W_d2, b_d2):
    n = x.shape[0]
    B = 512
    grid = (pl.cdiv(_NP, B),)

    def row_blk(cols):
        return pl.BlockSpec((B, cols), lambda i: (i, 0))

    def full(shape):
        return pl.BlockSpec(shape, lambda i: tuple(0 for _ in shape))

    return pl.pallas_call(
        _tc_body,
        grid=grid,
        in_specs=[
            row_blk(_W), row_blk(_W), row_blk(_D),
            full(W_l.shape), full(b_l.shape), full(W_r.shape),
            full(W_d1.shape), full(b_d1.shape),
            full(W_d2.shape), full(b_d2.shape),
        ],
        out_specs=row_blk(_D),
        out_shape=jax.ShapeDtypeStruct((n, _D), jnp.float32),
    )(a0, a1, x, W_l, b_l, W_r, W_d1, b_d1, W_d2, b_d2)


def kernel(x, edge_index, W_l, b_l, W_r, W_d1, b_d1, W_d2, b_d2):
    n = x.shape[0]
    e = edge_index.shape[1]
    f32 = jnp.float32

    src = edge_index[0].astype(jnp.int32)
    dst = edge_index[1].astype(jnp.int32)

    ones = jnp.ones((n, 1), f32)
    padc = jnp.zeros((n, _W - _HALF - 1), f32)
    padr = jnp.zeros((_NP - n, _W), f32)
    xflat = jnp.concatenate([
        jnp.concatenate([x[:, :_HALF], ones, padc], axis=1), padr,
        jnp.concatenate([x[:, _HALF:], ones, padc], axis=1), padr,
    ], axis=0)

    step = _TILES * _CH
    kk = 2 * _K
    nchunks = ((e + step - 1) // step + kk - 1) // kk * kk
    ep = nchunks * step
    nr = ep // _CH + _K
    srcp = jnp.concatenate(
        [src, jnp.zeros((nr * _CH - e,), jnp.int32)]).reshape(nr, _CH)
    dstp = jnp.concatenate(
        [dst, jnp.full((nr * _CH - e,), n, jnp.int32)]).reshape(nr, _CH)
    srcs2 = jnp.stack([srcp, srcp + _NP])
    zrows = jnp.zeros((_ROWS_PER_TILE, _W), f32)

    agg2 = _sc_segment_sum(xflat, srcs2, dstp, zrows, nchunks)

    b_l2 = b_l.reshape(1, -1)
    b_d12 = b_d1.reshape(1, -1)
    b_d22 = b_d2.reshape(1, -1)
    return _tc_dense(agg2[0], agg2[1], x, W_l, b_l2, W_r,
                     W_d1, b_d12, W_d2, b_d22)


# speedup vs baseline: 3.8834x; 1.1932x over previous
import functools
import jax
import jax.numpy as jnp
from jax import lax
from jax.experimental import pallas as pl
from jax.experimental.pallas import tpu as pltpu
from jax.experimental.pallas import tpu_sc as plsc

_N = 10000
_NP = 10016
_D = 256
_HALF = 128
_CH = 128
_K = 4
_TILES = 16
_ROWS_PER_TILE = _NP // _TILES
_DW = 16


def _sc_segment_sum(xr2, srcs2, dsts, zrows, zdeg, onesrow, nchunks):
    mesh = plsc.VectorSubcoreMesh(core_axis_name="c", subcore_axis_name="s")
    nsup = nchunks // _K

    @functools.partial(
        pl.kernel,
        out_type=(jax.ShapeDtypeStruct((2, _NP, _HALF), jnp.float32),
                  jax.ShapeDtypeStruct((_NP, _DW), jnp.float32)),
        mesh=mesh,
        compiler_params=pltpu.CompilerParams(use_tc_tiling_on_sc=False),
        scratch_types=[
            pltpu.VMEM((2, _K, _CH), jnp.int32),
            pltpu.VMEM((2, _K, _CH), jnp.int32),
            pltpu.VMEM((_CH, _HALF), jnp.float32),
            pltpu.VMEM((_CH, _HALF), jnp.float32),
            pltpu.VMEM((_CH, _DW), jnp.float32),
            pltpu.SemaphoreType.DMA,
            pltpu.SemaphoreType.DMA,
            pltpu.VMEM_SHARED((_NP, _HALF), jnp.float32),
            pltpu.VMEM_SHARED((_NP, _DW), jnp.float32),
        ],
    )
    def k(xr2_hbm, src_hbm, dst_hbm, z_hbm, zd_hbm, ones_hbm,
          agg_out, deg_out,
          src_sup, dst_sup, rows0, rows1, ones_v, sem0, sem1,
          agg_sh, deg_sh):
        cid = lax.axis_index("c")
        sid = lax.axis_index("s")
        row0 = sid * _ROWS_PER_TILE
        rows_v = (rows0, rows1)
        sems = (sem0, sem1)
        crow0 = sid * nchunks

        def load_sup(s, sb):
            pltpu.sync_copy(src_hbm.at[cid, pl.ds(crow0 + s * _K, _K)],
                            src_sup.at[sb])
            pltpu.sync_copy(dst_hbm.at[pl.ds(crow0 + s * _K, _K)],
                            dst_sup.at[sb])

        def fire(sb, j, b):
            pltpu.async_copy(xr2_hbm.at[src_sup.at[sb, j]],
                             rows_v[b], sems[b])

        def wait(sb, j, b):
            pltpu.make_async_copy(xr2_hbm.at[src_sup.at[sb, j]],
                                  rows_v[b], sems[b]).wait()

        pltpu.sync_copy(ones_hbm, ones_v)
        pltpu.sync_copy(z_hbm, agg_sh.at[pl.ds(row0, _ROWS_PER_TILE)])

        @pl.when(cid == 0)
        def _():
            pltpu.sync_copy(zd_hbm, deg_sh.at[pl.ds(row0, _ROWS_PER_TILE)])

        plsc.subcore_barrier()

        load_sup(0, 0)
        fire(0, 0, 0)

        def body(i, carry):
            for sp in range(2):
                s = 2 * i + sp
                sb = sp
                load_sup(s + 1, 1 - sb)
                for j in range(_K):
                    b = j % 2
                    jn, sbn = (j + 1, sb) if j + 1 < _K else (0, 1 - sb)
                    fire(sbn, jn, 1 - b)
                    wait(sb, j, b)
                    pltpu.sync_copy(rows_v[b],
                                    agg_sh.at[dst_sup.at[sb, j]], add=True)

                    @pl.when(cid == 0)
                    def _():
                        pltpu.sync_copy(ones_v,
                                        deg_sh.at[dst_sup.at[sb, j]],
                                        add=True)
            return carry

        lax.fori_loop(0, nsup // 2, body, 0)
        wait(0, 0, 0)

        plsc.subcore_barrier()
        pltpu.sync_copy(agg_sh.at[pl.ds(row0, _ROWS_PER_TILE)],
                        agg_out.at[cid, pl.ds(row0, _ROWS_PER_TILE)])

        @pl.when(cid == 0)
        def _():
            pltpu.sync_copy(deg_sh.at[pl.ds(row0, _ROWS_PER_TILE)],
                            deg_out.at[pl.ds(row0, _ROWS_PER_TILE)])

    return k(xr2, srcs2, dsts, zrows, zdeg, onesrow)


def _tc_body(a0, a1, dg, x, wl, bl, wr, wd1, bd1, wd2, bd2, o):
    deg = jnp.clip(dg[:, :1], 1.0, None)
    agg = jnp.concatenate([a0[...], a1[...]], axis=1) / deg
    h = jnp.dot(agg, wl[...], preferred_element_type=jnp.float32)
    h += jnp.dot(x[...], wr[...], preferred_element_type=jnp.float32)
    h = jnp.maximum(h + bl[...], 0.0)
    hd = jnp.dot(h, wd1[...], preferred_element_type=jnp.float32)
    hd = jnp.maximum(hd + bd1[...], 0.0)
    out = jnp.dot(hd, wd2[...], preferred_element_type=jnp.float32)
    o[...] = out + bd2[...]


def _tc_dense(a0, a1, dg, x, W_l, b_l, W_r, W_d1, b_d1, W_d2, b_d2):
    n = x.shape[0]
    B = 512
    grid = (pl.cdiv(_NP, B),)

    def row_blk(cols):
        return pl.BlockSpec((B, cols), lambda i: (i, 0))

    def full(shape):
        return pl.BlockSpec(shape, lambda i: tuple(0 for _ in shape))

    return pl.pallas_call(
        _tc_body,
        grid=grid,
        in_specs=[
            row_blk(_HALF), row_blk(_HALF), row_blk(_DW), row_blk(_D),
            full(W_l.shape), full(b_l.shape), full(W_r.shape),
            full(W_d1.shape), full(b_d1.shape),
            full(W_d2.shape), full(b_d2.shape),
        ],
        out_specs=row_blk(_D),
        out_shape=jax.ShapeDtypeStruct((n, _D), jnp.float32),
    )(a0, a1, dg, x, W_l, b_l, W_r, W_d1, b_d1, W_d2, b_d2)


def kernel(x, edge_index, W_l, b_l, W_r, W_d1, b_d1, W_d2, b_d2):
    n = x.shape[0]
    e = edge_index.shape[1]
    f32 = jnp.float32

    src = edge_index[0].astype(jnp.int32)
    dst = edge_index[1].astype(jnp.int32)

    xr2 = x.reshape(2 * n, _HALF)

    step = _TILES * _CH
    kk = 2 * _K
    nchunks = ((e + step - 1) // step + kk - 1) // kk * kk
    ep = nchunks * step
    nr = ep // _CH + _K
    srcp = jnp.concatenate(
        [src, jnp.zeros((nr * _CH - e,), jnp.int32)]).reshape(nr, _CH)
    dstp = jnp.concatenate(
        [dst, jnp.full((nr * _CH - e,), n, jnp.int32)]).reshape(nr, _CH)
    src2 = 2 * srcp
    srcs2 = jnp.stack([src2, src2 + 1])
    zrows = jnp.zeros((_ROWS_PER_TILE, _HALF), f32)
    zdeg = jnp.zeros((_ROWS_PER_TILE, _DW), f32)
    onesrow = jnp.ones((_CH, _DW), f32)

    agg2, deg = _sc_segment_sum(xr2, srcs2, dstp, zrows, zdeg, onesrow,
                                nchunks)

    b_l2 = b_l.reshape(1, -1)
    b_d12 = b_d1.reshape(1, -1)
    b_d22 = b_d2.reshape(1, -1)
    return _tc_dense(agg2[0], agg2[1], deg, x, W_l, b_l2, W_r,
                     W_d1, b_d12, W_d2, b_d22)
